# baseline (speedup 1.0000x reference)
import jax
import jax.numpy as jnp
from jax.experimental import pallas as pl
from jax.experimental.pallas import tpu as pltpu

_N = 20
_IN = 256
_OUT = 2
_K = 2
_EMB = 4
_CIN = _N * _IN
_CO = 4 * _N


def _prep_kernel(e_ref, et_ref, pt_ref, bexp_ref, a_ref, brow_ref):
    g = jnp.maximum(
        jnp.dot(e_ref[...], et_ref[...], preferred_element_type=jnp.float32), 0.0
    )
    g = g - jnp.max(g, axis=0, keepdims=True)
    ex = jnp.exp(g)
    st = ex / jnp.sum(ex, axis=0, keepdims=True)

    row_i = jax.lax.broadcasted_iota(jnp.int32, (_N, _N), 0)
    col_i = jax.lax.broadcasted_iota(jnp.int32, (_N, _N), 1)
    eye = jnp.where(row_i == col_i, 1.0, 0.0).astype(jnp.float32)

    for m in range(_N):
        acc = jnp.zeros((_IN, _CO), jnp.float32)
        for k in range(_K):
            skt = eye if k == 0 else st
            srow = skt[m:m + 1, :]
            for d in range(_EMB):
                c20 = srow * et_ref[d:d + 1, :]
                c80 = jnp.concatenate([c20, c20, c20, c20], axis=1)
                blk = (k * _EMB + d) * _IN
                acc = acc + pt_ref[blk:blk + _IN, :] * c80
        a_ref[m * _IN:(m + 1) * _IN, :] = acc

    brow = jnp.zeros((1, _CO), jnp.float32)
    for d in range(_EMB):
        er = et_ref[d:d + 1, :]
        er4 = jnp.concatenate([er, er, er, er], axis=1)
        brow = brow + er4 * bexp_ref[d:d + 1, :]
    brow_ref[...] = jnp.broadcast_to(brow, (8, _CO))


def _block_body(x_ref, a_ref, b_ref, wl_ref, bl_ref, y_ref):
    out = jnp.dot(x_ref[:, 0, :], a_ref[0:_IN, :],
                  preferred_element_type=jnp.float32,
                  precision=jax.lax.Precision.HIGHEST)
    for m in range(1, _N):
        out = out + jnp.dot(x_ref[:, m, :], a_ref[m * _IN:(m + 1) * _IN, :],
                            preferred_element_type=jnp.float32,
                            precision=jax.lax.Precision.HIGHEST)
    out = out + b_ref[0:1, :]
    r0 = out[:, 0:_N]
    r1 = out[:, _N:2 * _N]
    h0 = out[:, 2 * _N:3 * _N]
    h1 = out[:, 3 * _N:4 * _N]
    hn0 = (1.0 - jax.nn.sigmoid(r0)) * jnp.tanh(h0)
    hn1 = (1.0 - jax.nn.sigmoid(r1)) * jnp.tanh(h1)
    y = (jnp.maximum(hn0, 0.0) * wl_ref[0:1, 0:1]
         + jnp.maximum(hn1, 0.0) * wl_ref[0:1, 1:2]
         + bl_ref[0:1, 0:1])
    y_ref[...] = y


def _main_kernel(x_ref, a_ref, b_ref, wl_ref, bl_ref, y_ref):
    _block_body(x_ref, a_ref, b_ref, wl_ref, bl_ref, y_ref)


def kernel(x, e, _, Wg, bg, Wu, bu, Wl, bl):
    bsz = x.shape[0]

    p = jnp.concatenate([Wg[:, :, :_IN, _OUT:], Wu[:, :, :_IN, :]], axis=-1)
    p = jnp.transpose(p, (1, 0, 2, 3)).reshape(_K * _EMB * _IN, 4)
    pt = jnp.repeat(p, _N, axis=1)
    et = e.T
    bexp = jnp.repeat(jnp.concatenate([bg[:, _OUT:], bu], axis=1), _N, axis=1)

    a_mat, brow = pl.pallas_call(
        _prep_kernel,
        out_shape=(
            jax.ShapeDtypeStruct((_CIN, _CO), jnp.float32),
            jax.ShapeDtypeStruct((8, _CO), jnp.float32),
        ),
    )(e, et, pt, bexp)

    bb = 1024
    while bsz % bb:
        bb //= 2
    y2 = pl.pallas_call(
        _main_kernel,
        grid=(bsz // bb,),
        in_specs=[
            pl.BlockSpec((bb, _N, _IN), lambda i: (i, 0, 0)),
            pl.BlockSpec((_CIN, _CO), lambda i: (0, 0)),
            pl.BlockSpec((8, _CO), lambda i: (0, 0)),
            pl.BlockSpec((1, 2), lambda i: (0, 0)),
            pl.BlockSpec((1, 1), lambda i: (0, 0)),
        ],
        out_specs=pl.BlockSpec((bb, _N), lambda i: (i, 0)),
        out_shape=jax.ShapeDtypeStruct((bsz, _N), jnp.float32),
        compiler_params=pltpu.CompilerParams(dimension_semantics=("arbitrary",)),
    )(x, a_mat, brow, Wl, bl.reshape(1, 1))
    return y2.reshape(bsz, _N, 1)


# speedup vs baseline: 1.4976x; 1.4976x over previous
import jax
import jax.numpy as jnp
from jax.experimental import pallas as pl
from jax.experimental.pallas import tpu as pltpu

_N = 20
_IN = 256
_OUT = 2
_K = 2
_EMB = 4
_CIN = _N * _IN
_CO = 4 * _N


def _prep_kernel(e_ref, et_ref, pt_ref, bexp_ref, a_ref, brow_ref):
    g = jnp.maximum(
        jnp.dot(e_ref[...], et_ref[...], preferred_element_type=jnp.float32), 0.0
    )
    g = g - jnp.max(g, axis=0, keepdims=True)
    ex = jnp.exp(g)
    st = ex / jnp.sum(ex, axis=0, keepdims=True)

    row_i = jax.lax.broadcasted_iota(jnp.int32, (_N, _N), 0)
    col_i = jax.lax.broadcasted_iota(jnp.int32, (_N, _N), 1)
    eye = jnp.where(row_i == col_i, 1.0, 0.0).astype(jnp.float32)

    for m in range(_N):
        acc = jnp.zeros((_IN, _CO), jnp.float32)
        for k in range(_K):
            skt = eye if k == 0 else st
            srow = skt[m:m + 1, :]
            for d in range(_EMB):
                c20 = srow * et_ref[d:d + 1, :]
                c80 = jnp.concatenate([c20, c20, c20, c20], axis=1)
                blk = (k * _EMB + d) * _IN
                acc = acc + pt_ref[blk:blk + _IN, :] * c80
        a_ref[m * _IN:(m + 1) * _IN, :] = acc

    brow = jnp.zeros((1, _CO), jnp.float32)
    for d in range(_EMB):
        er = et_ref[d:d + 1, :]
        er4 = jnp.concatenate([er, er, er, er], axis=1)
        brow = brow + er4 * bexp_ref[d:d + 1, :]
    brow_ref[...] = jnp.broadcast_to(brow, (8, _CO))


def _block_body(x_ref, a_ref, b_ref, wl_ref, bl_ref, y_ref):
    out = jnp.dot(x_ref[:, 0, :], a_ref[0:_IN, :],
                  preferred_element_type=jnp.float32)
    for m in range(1, _N):
        out = out + jnp.dot(x_ref[:, m, :], a_ref[m * _IN:(m + 1) * _IN, :],
                            preferred_element_type=jnp.float32)
    out = out + b_ref[0:1, :]
    r0 = out[:, 0:_N]
    r1 = out[:, _N:2 * _N]
    h0 = out[:, 2 * _N:3 * _N]
    h1 = out[:, 3 * _N:4 * _N]
    hn0 = (1.0 - jax.nn.sigmoid(r0)) * jnp.tanh(h0)
    hn1 = (1.0 - jax.nn.sigmoid(r1)) * jnp.tanh(h1)
    y = (jnp.maximum(hn0, 0.0) * wl_ref[0:1, 0:1]
         + jnp.maximum(hn1, 0.0) * wl_ref[0:1, 1:2]
         + bl_ref[0:1, 0:1])
    y_ref[...] = y


def _main_kernel(x_ref, a_ref, b_ref, wl_ref, bl_ref, y_ref):
    _block_body(x_ref, a_ref, b_ref, wl_ref, bl_ref, y_ref)


def kernel(x, e, _, Wg, bg, Wu, bu, Wl, bl):
    bsz = x.shape[0]

    p = jnp.concatenate([Wg[:, :, :_IN, _OUT:], Wu[:, :, :_IN, :]], axis=-1)
    p = jnp.transpose(p, (1, 0, 2, 3)).reshape(_K * _EMB * _IN, 4)
    pt = jnp.repeat(p, _N, axis=1)
    et = e.T
    bexp = jnp.repeat(jnp.concatenate([bg[:, _OUT:], bu], axis=1), _N, axis=1)

    a_mat, brow = pl.pallas_call(
        _prep_kernel,
        out_shape=(
            jax.ShapeDtypeStruct((_CIN, _CO), jnp.float32),
            jax.ShapeDtypeStruct((8, _CO), jnp.float32),
        ),
    )(e, et, pt, bexp)

    bb = 1024
    while bsz % bb:
        bb //= 2
    y2 = pl.pallas_call(
        _main_kernel,
        grid=(bsz // bb,),
        in_specs=[
            pl.BlockSpec((bb, _N, _IN), lambda i: (i, 0, 0)),
            pl.BlockSpec((_CIN, _CO), lambda i: (0, 0)),
            pl.BlockSpec((8, _CO), lambda i: (0, 0)),
            pl.BlockSpec((1, 2), lambda i: (0, 0)),
            pl.BlockSpec((1, 1), lambda i: (0, 0)),
        ],
        out_specs=pl.BlockSpec((bb, _N), lambda i: (i, 0)),
        out_shape=jax.ShapeDtypeStruct((bsz, _N), jnp.float32),
        compiler_params=pltpu.CompilerParams(dimension_semantics=("arbitrary",)),
    )(x, a_mat, brow, Wl, bl.reshape(1, 1))
    return y2.reshape(bsz, _N, 1)


# speedup vs baseline: 1.5123x; 1.0098x over previous
import jax
import jax.numpy as jnp
from jax.experimental import pallas as pl
from jax.experimental.pallas import tpu as pltpu

_N = 20
_IN = 256
_OUT = 2
_K = 2
_EMB = 4
_CIN = _N * _IN
_CO = 4 * _N


def _prep_kernel(e_ref, et_ref, pt_ref, bexp_ref, a_ref, brow_ref):
    g = jnp.maximum(
        jnp.dot(e_ref[...], et_ref[...], preferred_element_type=jnp.float32), 0.0
    )
    g = g - jnp.max(g, axis=0, keepdims=True)
    ex = jnp.exp(g)
    st = ex / jnp.sum(ex, axis=0, keepdims=True)

    row_i = jax.lax.broadcasted_iota(jnp.int32, (_N, _N), 0)
    col_i = jax.lax.broadcasted_iota(jnp.int32, (_N, _N), 1)
    eye = jnp.where(row_i == col_i, 1.0, 0.0).astype(jnp.float32)

    for m in range(_N):
        acc = jnp.zeros((_IN, _CO), jnp.float32)
        for k in range(_K):
            skt = eye if k == 0 else st
            srow = skt[m:m + 1, :]
            for d in range(_EMB):
                c20 = srow * et_ref[d:d + 1, :]
                c80 = jnp.concatenate([c20, c20, c20, c20], axis=1)
                blk = (k * _EMB + d) * _IN
                acc = acc + pt_ref[blk:blk + _IN, :] * c80
        a_ref[m * _IN:(m + 1) * _IN, :] = acc

    brow = jnp.zeros((1, _CO), jnp.float32)
    for d in range(_EMB):
        er = et_ref[d:d + 1, :]
        er4 = jnp.concatenate([er, er, er, er], axis=1)
        brow = brow + er4 * bexp_ref[d:d + 1, :]
    brow_ref[...] = jnp.broadcast_to(brow, (8, _CO))


def _main_kernel(x_ref, a_ref, b_ref, wl_ref, bl_ref, y_ref):
    out = jnp.dot(x_ref[:, 0, :], a_ref[0:_IN, :],
                  preferred_element_type=jnp.float32)
    for m in range(1, _N):
        out = out + jnp.dot(x_ref[:, m, :], a_ref[m * _IN:(m + 1) * _IN, :],
                            preferred_element_type=jnp.float32)
    out = out + b_ref[0:1, :]
    r0 = out[:, 0:_N]
    r1 = out[:, _N:2 * _N]
    h0 = out[:, 2 * _N:3 * _N]
    h1 = out[:, 3 * _N:4 * _N]
    hn0 = (1.0 - jax.nn.sigmoid(r0)) * jnp.tanh(h0)
    hn1 = (1.0 - jax.nn.sigmoid(r1)) * jnp.tanh(h1)
    y = (jnp.maximum(hn0, 0.0) * wl_ref[0:1, 0:1]
         + jnp.maximum(hn1, 0.0) * wl_ref[0:1, 1:2]
         + bl_ref[0:1, 0:1])
    y_ref[...] = y


def kernel(x, e, _, Wg, bg, Wu, bu, Wl, bl):
    bsz = x.shape[0]

    p = jnp.concatenate([Wg[:, :, :_IN, _OUT:], Wu[:, :, :_IN, :]], axis=-1)
    p = jnp.transpose(p, (1, 0, 2, 3)).reshape(_K * _EMB * _IN, 4)
    pt = jnp.repeat(p, _N, axis=1)
    et = e.T
    bexp = jnp.repeat(jnp.concatenate([bg[:, _OUT:], bu], axis=1), _N, axis=1)

    a_mat, brow = pl.pallas_call(
        _prep_kernel,
        out_shape=(
            jax.ShapeDtypeStruct((_CIN, _CO), jnp.float32),
            jax.ShapeDtypeStruct((8, _CO), jnp.float32),
        ),
    )(e, et, pt, bexp)

    bb = 1024
    while bsz % bb:
        bb //= 2
    y2 = pl.pallas_call(
        _main_kernel,
        grid=(bsz // bb,),
        in_specs=[
            pl.BlockSpec((bb, _N, _IN), lambda i: (i, 0, 0)),
            pl.BlockSpec((_CIN, _CO), lambda i: (0, 0)),
            pl.BlockSpec((8, _CO), lambda i: (0, 0)),
            pl.BlockSpec((1, 2), lambda i: (0, 0)),
            pl.BlockSpec((1, 1), lambda i: (0, 0)),
        ],
        out_specs=pl.BlockSpec((bb, _N), lambda i: (i, 0)),
        out_shape=jax.ShapeDtypeStruct((bsz, _N), jnp.float32),
        compiler_params=pltpu.CompilerParams(dimension_semantics=("arbitrary",)),
    )(x, a_mat, brow, Wl, bl.reshape(1, 1))
    return y2.reshape(bsz, _N, 1)
